# Initial kernel scaffold; baseline (speedup 1.0000x reference)
import functools

import jax
import jax.numpy as jnp
from jax import lax
from jax.experimental import pallas as pl
from jax.experimental.pallas import tpu as pltpu
from jax.experimental.pallas import tpu_sc as plsc

NC = 2
NS = 16
LANES = 16
K = 80


def _sc_degree(NPAD, RPT, E_PER, ITERS):
    mesh = plsc.VectorSubcoreMesh(core_axis_name="c", subcore_axis_name="s")

    @functools.partial(
        pl.kernel,
        out_type=jax.ShapeDtypeStruct((NC, NS, RPT), jnp.float32),
        mesh=mesh,
        scratch_types=[
            pltpu.VMEM((K,), jnp.int32),
            pltpu.VMEM((K,), jnp.float32),
            pltpu.VMEM_SHARED((NPAD,), jnp.float32),
        ],
    )
    def deg_kernel(edge_hbm, zrow_hbm, out_hbm, idx_v, ones_v, deg_sh):
        c = lax.axis_index("c")
        s = lax.axis_index("s")
        wid = s * NC + c
        pltpu.sync_copy(zrow_hbm, deg_sh.at[pl.ds(s * RPT, RPT)])
        for j in range(K // LANES):
            ones_v[pl.ds(j * LANES, LANES)] = jnp.ones((LANES,), jnp.float32)
        plsc.subcore_barrier()

        def step(i, carry):
            base = wid * E_PER + i * K
            pltpu.sync_copy(edge_hbm.at[1, pl.ds(base, K)], idx_v)
            pltpu.sync_copy(ones_v, deg_sh.at[idx_v], add=True)
            return carry

        lax.fori_loop(0, ITERS, step, 0)
        plsc.subcore_barrier()
        pltpu.sync_copy(deg_sh.at[pl.ds(s * RPT, RPT)], out_hbm.at[c, s])

    return deg_kernel


def _sc_aggregate(NPAD, RPT, E_PER, ITERS, D):
    mesh = plsc.VectorSubcoreMesh(core_axis_name="c", subcore_axis_name="s")

    @functools.partial(
        pl.kernel,
        out_type=jax.ShapeDtypeStruct((NC, NS, RPT, D), jnp.float32),
        mesh=mesh,
        scratch_types=[
            pltpu.VMEM((2, K), jnp.int32),
            pltpu.VMEM((K, D), jnp.float32),
            pltpu.VMEM_SHARED((NPAD, D), jnp.float32),
            pltpu.SemaphoreType.DMA,
        ],
    )
    def agg_kernel(h_hbm, edge_hbm, zeros_hbm, out_hbm, idx_v, rows_v, accum_sh, sem):
        c = lax.axis_index("c")
        s = lax.axis_index("s")
        wid = s * NC + c
        pltpu.sync_copy(zeros_hbm, accum_sh.at[pl.ds(s * RPT, RPT)])
        plsc.subcore_barrier()

        def step(i, carry):
            base = wid * E_PER + i * K
            pltpu.sync_copy(edge_hbm.at[0, pl.ds(base, K)], idx_v.at[0])
            pltpu.sync_copy(edge_hbm.at[1, pl.ds(base, K)], idx_v.at[1])
            pltpu.async_copy(h_hbm.at[idx_v.at[0]], rows_v, sem).wait()
            pltpu.sync_copy(rows_v, accum_sh.at[idx_v.at[1]], add=True)
            return carry

        lax.fori_loop(0, ITERS, step, 0)
        plsc.subcore_barrier()
        pltpu.sync_copy(accum_sh.at[pl.ds(s * RPT, RPT)], out_hbm.at[c, s])

    return agg_kernel


def _norm_from(dp0, dp1):
    deg = dp0 + dp1
    return jnp.where(deg > 0, lax.rsqrt(jnp.maximum(deg, 1.0)), 0.0)


def _tc_linear(N, D, NPAD, Bn):
    def body(x_ref, w_ref, b_ref, dp_ref, out_ref):
        norm = _norm_from(dp_ref[0, :], dp_ref[1, :])
        h = lax.dot_general(x_ref[...], w_ref[...], (((1,), (1,)), ((), ())),
                            preferred_element_type=jnp.float32)
        out_ref[...] = (h + b_ref[...]) * norm[:, None]

    return pl.pallas_call(
        body,
        grid=(N // Bn,),
        in_specs=[
            pl.BlockSpec((Bn, D), lambda i: (i, 0)),
            pl.BlockSpec((D, D), lambda i: (0, 0)),
            pl.BlockSpec((1, D), lambda i: (0, 0)),
            pl.BlockSpec((NC, Bn), lambda i: (0, i)),
        ],
        out_specs=pl.BlockSpec((Bn, D), lambda i: (i, 0)),
        out_shape=jax.ShapeDtypeStruct((N, D), jnp.float32),
    )


def _tc_finish(N, D, NPAD, Bn):
    def body(ap_ref, dp_ref, out_ref):
        norm = _norm_from(dp_ref[0, :], dp_ref[1, :])
        a = ap_ref[0] + ap_ref[1]
        out_ref[...] = jnp.maximum(a * norm[:, None], 0.0)

    return pl.pallas_call(
        body,
        grid=(N // Bn,),
        in_specs=[
            pl.BlockSpec((NC, Bn, D), lambda i: (0, i, 0)),
            pl.BlockSpec((NC, Bn), lambda i: (0, i)),
        ],
        out_specs=pl.BlockSpec((Bn, D), lambda i: (i, 0)),
        out_shape=jax.ShapeDtypeStruct((N, D), jnp.float32),
    )


def kernel(features, edge_index, W, b):
    N, D = features.shape
    E = edge_index.shape[1]
    NW = NC * NS
    E_PER = E // NW
    ITERS = E_PER // K
    NPAD = ((N + NS * 8 - 1) // (NS * 8)) * (NS * 8)
    RPT = NPAD // NS
    Bn = 1000

    edge = edge_index.astype(jnp.int32)
    zrow = jnp.zeros((RPT,), jnp.float32)
    zeros2d = jnp.zeros((RPT, D), jnp.float32)

    degp = _sc_degree(NPAD, RPT, E_PER, ITERS)(edge, zrow)
    dp = degp.reshape(NC, NPAD)
    h = _tc_linear(N, D, NPAD, Bn)(features, W, b.reshape(1, D), dp)
    accp = _sc_aggregate(NPAD, RPT, E_PER, ITERS, D)(h, edge, zeros2d)
    ap = accp.reshape(NC, NPAD, D)
    out = _tc_finish(N, D, NPAD, Bn)(ap, dp)
    return out


# speedup vs baseline: 23.7968x; 23.7968x over previous
import functools

import jax
import jax.numpy as jnp
from jax import lax
from jax.experimental import pallas as pl
from jax.experimental.pallas import tpu as pltpu
from jax.experimental.pallas import tpu_sc as plsc

NC = 2
NS = 16
LANES = 16
K = 80
RB = 3
IB = 4


def _sc_degree(NPAD, RPT, ITERS):
    mesh = plsc.VectorSubcoreMesh(core_axis_name="c", subcore_axis_name="s")

    @functools.partial(
        pl.kernel,
        out_type=jax.ShapeDtypeStruct((NC, NS, RPT), jnp.float32),
        mesh=mesh,
        scratch_types=[
            pltpu.VMEM((ITERS, K), jnp.int32),
            pltpu.VMEM((K,), jnp.float32),
            pltpu.VMEM((RPT,), jnp.float32),
            pltpu.VMEM_SHARED((NPAD,), jnp.float32),
            pltpu.SemaphoreType.DMA,
        ],
    )
    def deg_kernel(dst_hbm, out_hbm, idxb, ones_v, zvec_v, deg_sh, ssem):
        c = lax.axis_index("c")
        s = lax.axis_index("s")
        wid = s * NC + c

        pltpu.sync_copy(dst_hbm.at[wid], idxb)

        def zfill(i, carry):
            zvec_v[pl.ds(i * LANES, LANES)] = jnp.zeros((LANES,), jnp.float32)
            return carry

        lax.fori_loop(0, RPT // LANES, zfill, 0)
        pltpu.sync_copy(zvec_v, deg_sh.at[pl.ds(s * RPT, RPT)])
        for j in range(K // LANES):
            ones_v[pl.ds(j * LANES, LANES)] = jnp.ones((LANES,), jnp.float32)
        plsc.subcore_barrier()

        def fire(i, carry):
            pltpu.async_copy(ones_v, deg_sh.at[idxb.at[i]], ssem, add=True)
            return carry

        lax.fori_loop(0, ITERS, fire, 0)

        def drain(i, carry):
            pltpu.make_async_copy(ones_v, deg_sh.at[idxb.at[i]], ssem).wait()
            return carry

        lax.fori_loop(0, ITERS, drain, 0)
        plsc.subcore_barrier()
        pltpu.sync_copy(deg_sh.at[pl.ds(s * RPT, RPT)], out_hbm.at[c, s])

    return deg_kernel


def _sc_aggregate(NPAD, RPT, ITERS, D):
    mesh = plsc.VectorSubcoreMesh(core_axis_name="c", subcore_axis_name="s")

    @functools.partial(
        pl.kernel,
        out_type=jax.ShapeDtypeStruct((NC, NS, RPT, D), jnp.float32),
        mesh=mesh,
        scratch_types=[
            pltpu.VMEM((IB, 2, K), jnp.int32),
            pltpu.VMEM((RB, K, D), jnp.float32),
            pltpu.VMEM_SHARED((NPAD, D), jnp.float32),
            pltpu.SemaphoreType.DMA((IB,)),
            pltpu.SemaphoreType.DMA((RB,)),
            pltpu.SemaphoreType.DMA((RB,)),
        ],
    )
    def agg_kernel(h_hbm, e_hbm, out_hbm, ibuf, rows_v, accum_sh, isem, gsem, ssem):
        c = lax.axis_index("c")
        s = lax.axis_index("s")
        wid = s * NC + c

        def zfill(r, carry):
            for j in range(D // LANES):
                rows_v[0, r, pl.ds(j * LANES, LANES)] = jnp.zeros((LANES,), jnp.float32)
            return carry

        lax.fori_loop(0, K, zfill, 0)

        def zcopy(k, carry):
            pltpu.sync_copy(rows_v.at[0], accum_sh.at[pl.ds(s * RPT + k * K, K)])
            return carry

        lax.fori_loop(0, RPT // K, zcopy, 0)
        plsc.subcore_barrier()

        pltpu.sync_copy(e_hbm.at[wid, 0], ibuf.at[0])
        pltpu.async_copy(h_hbm.at[ibuf.at[0, 0]], rows_v.at[0], gsem.at[0])
        pltpu.async_copy(e_hbm.at[wid, 1], ibuf.at[1], isem.at[1])

        def step(i, carry):
            p = lax.rem(i, RB)
            q = lax.rem(i + 1, RB)
            pi = lax.rem(i, IB)
            qi = lax.rem(i + 1, IB)
            ni = lax.rem(i + 2, IB)

            @pl.when(i + 1 < ITERS)
            def _():
                pltpu.make_async_copy(e_hbm.at[wid, i + 1], ibuf.at[qi],
                                      isem.at[qi]).wait()

                @pl.when(i >= 2)
                def _():
                    pltpu.make_async_copy(rows_v.at[q],
                                          accum_sh.at[ibuf.at[qi, 1]],
                                          ssem.at[q]).wait()

                pltpu.async_copy(h_hbm.at[ibuf.at[qi, 0]], rows_v.at[q],
                                 gsem.at[q])

            pltpu.make_async_copy(h_hbm.at[ibuf.at[pi, 0]], rows_v.at[p],
                                  gsem.at[p]).wait()
            pltpu.async_copy(rows_v.at[p], accum_sh.at[ibuf.at[pi, 1]],
                             ssem.at[p], add=True)

            @pl.when(i + 2 < ITERS)
            def _():
                pltpu.async_copy(e_hbm.at[wid, i + 2], ibuf.at[ni], isem.at[ni])

            return carry

        lax.fori_loop(0, ITERS, step, 0)

        def sdrain(j, carry):
            r = lax.rem(j, RB)
            pltpu.make_async_copy(rows_v.at[r], accum_sh.at[ibuf.at[0, 1]],
                                  ssem.at[r]).wait()
            return carry

        lax.fori_loop(0, RB, sdrain, 0)
        plsc.subcore_barrier()
        pltpu.sync_copy(accum_sh.at[pl.ds(s * RPT, RPT)], out_hbm.at[c, s])

    return agg_kernel


def _norm_from(dp0, dp1):
    deg = dp0 + dp1
    return jnp.where(deg > 0, lax.rsqrt(jnp.maximum(deg, 1.0)), 0.0)


def _tc_linear(N, D, NPAD, Bn):
    def body(x_ref, w_ref, b_ref, dp_ref, out_ref):
        norm = _norm_from(dp_ref[:, 0], dp_ref[:, 1])
        h = lax.dot_general(x_ref[...], w_ref[...], (((1,), (1,)), ((), ())),
                            preferred_element_type=jnp.float32)
        out_ref[...] = (h + b_ref[...]) * norm[:, None]

    return pl.pallas_call(
        body,
        grid=(N // Bn,),
        in_specs=[
            pl.BlockSpec((Bn, D), lambda i: (i, 0)),
            pl.BlockSpec((D, D), lambda i: (0, 0)),
            pl.BlockSpec((1, D), lambda i: (0, 0)),
            pl.BlockSpec((Bn, NC), lambda i: (i, 0)),
        ],
        out_specs=pl.BlockSpec((Bn, D), lambda i: (i, 0)),
        out_shape=jax.ShapeDtypeStruct((N, D), jnp.float32),
    )


def _tc_finish(N, D, NPAD, Bn):
    def body(ap_ref, dp_ref, out_ref):
        norm = _norm_from(dp_ref[:, 0], dp_ref[:, 1])
        a = ap_ref[0] + ap_ref[1]
        out_ref[...] = jnp.maximum(a * norm[:, None], 0.0)

    return pl.pallas_call(
        body,
        grid=(N // Bn,),
        in_specs=[
            pl.BlockSpec((NC, Bn, D), lambda i: (0, i, 0)),
            pl.BlockSpec((Bn, NC), lambda i: (i, 0)),
        ],
        out_specs=pl.BlockSpec((Bn, D), lambda i: (i, 0)),
        out_shape=jax.ShapeDtypeStruct((N, D), jnp.float32),
    )


def kernel(features, edge_index, W, b):
    N, D = features.shape
    E = edge_index.shape[1]
    NW = NC * NS
    E_PER = E // NW
    ITERS = E_PER // K
    NPAD = ((N + NS * LANES - 1) // (NS * LANES)) * (NS * LANES)
    RPT = NPAD // NS
    Bn = 1000

    edge = edge_index.astype(jnp.int32)
    dst3 = edge[1].reshape(NW, ITERS, K)
    e4 = edge.reshape(2, NW, ITERS, K).transpose(1, 2, 0, 3)

    degp = _sc_degree(NPAD, RPT, ITERS)(dst3)
    dp = degp.reshape(NC, NPAD).T
    h = _tc_linear(N, D, NPAD, Bn)(features, W, b.reshape(1, D), dp)
    accp = _sc_aggregate(NPAD, RPT, ITERS, D)(h, e4)
    ap = accp.reshape(NC, NPAD, D)
    out = _tc_finish(N, D, NPAD, Bn)(ap, dp)
    return out


# speedup vs baseline: 27.3187x; 1.1480x over previous
import functools

import jax
import jax.numpy as jnp
from jax import lax
from jax.experimental import pallas as pl
from jax.experimental.pallas import tpu as pltpu
from jax.experimental.pallas import tpu_sc as plsc

NC = 2
NS = 16
LANES = 16
K = 80
RB = 4
IB = 5


def _sc_degree(NPAD, RPT, ITERS):
    mesh = plsc.VectorSubcoreMesh(core_axis_name="c", subcore_axis_name="s")

    @functools.partial(
        pl.kernel,
        out_type=jax.ShapeDtypeStruct((NC, NS, RPT), jnp.float32),
        mesh=mesh,
        scratch_types=[
            pltpu.VMEM((ITERS, K), jnp.int32),
            pltpu.VMEM((K,), jnp.float32),
            pltpu.VMEM((RPT,), jnp.float32),
            pltpu.VMEM_SHARED((NPAD,), jnp.float32),
            pltpu.SemaphoreType.DMA,
        ],
    )
    def deg_kernel(dst_hbm, out_hbm, idxb, ones_v, zvec_v, deg_sh, ssem):
        c = lax.axis_index("c")
        s = lax.axis_index("s")
        wid = s * NC + c

        pltpu.sync_copy(dst_hbm.at[wid], idxb)

        def zfill(i, carry):
            zvec_v[pl.ds(i * LANES, LANES)] = jnp.zeros((LANES,), jnp.float32)
            return carry

        lax.fori_loop(0, RPT // LANES, zfill, 0)
        pltpu.sync_copy(zvec_v, deg_sh.at[pl.ds(s * RPT, RPT)])
        for j in range(K // LANES):
            ones_v[pl.ds(j * LANES, LANES)] = jnp.ones((LANES,), jnp.float32)
        plsc.subcore_barrier()

        def fire(i, carry):
            pltpu.async_copy(ones_v, deg_sh.at[idxb.at[i]], ssem, add=True)
            return carry

        lax.fori_loop(0, ITERS, fire, 0)

        def drain(i, carry):
            pltpu.make_async_copy(ones_v, deg_sh.at[idxb.at[i]], ssem).wait()
            return carry

        lax.fori_loop(0, ITERS, drain, 0)
        plsc.subcore_barrier()
        pltpu.sync_copy(deg_sh.at[pl.ds(s * RPT, RPT)], out_hbm.at[c, s])

    return deg_kernel


def _sc_aggregate(NPAD, RPT, ITERS, D):
    mesh = plsc.VectorSubcoreMesh(core_axis_name="c", subcore_axis_name="s")

    @functools.partial(
        pl.kernel,
        out_type=jax.ShapeDtypeStruct((NC, NS, RPT, D), jnp.float32),
        mesh=mesh,
        scratch_types=[
            pltpu.VMEM((IB, 2, K), jnp.int32),
            pltpu.VMEM((RB, K, D), jnp.float32),
            pltpu.VMEM_SHARED((NPAD, D), jnp.float32),
            pltpu.SemaphoreType.DMA((IB,)),
            pltpu.SemaphoreType.DMA((RB,)),
            pltpu.SemaphoreType.DMA((RB,)),
        ],
    )
    def agg_kernel(h_hbm, e_hbm, out_hbm, ibuf, rows_v, accum_sh, isem, gsem, ssem):
        c = lax.axis_index("c")
        s = lax.axis_index("s")
        wid = s * NC + c

        def zfill(r, carry):
            for j in range(D // LANES):
                rows_v[0, r, pl.ds(j * LANES, LANES)] = jnp.zeros((LANES,), jnp.float32)
            return carry

        lax.fori_loop(0, K, zfill, 0)

        def zcopy(k, carry):
            pltpu.sync_copy(rows_v.at[0], accum_sh.at[pl.ds(s * RPT + k * K, K)])
            return carry

        lax.fori_loop(0, RPT // K, zcopy, 0)
        plsc.subcore_barrier()

        pltpu.sync_copy(e_hbm.at[wid, 0], ibuf.at[0])
        pltpu.sync_copy(e_hbm.at[wid, 1], ibuf.at[1])
        pltpu.async_copy(h_hbm.at[ibuf.at[0, 0]], rows_v.at[0], gsem.at[0])
        pltpu.async_copy(h_hbm.at[ibuf.at[1, 0]], rows_v.at[1], gsem.at[1])
        pltpu.async_copy(e_hbm.at[wid, 2], ibuf.at[2], isem.at[2])

        def step(i, carry):
            p = lax.rem(i, RB)
            g2 = lax.rem(i + 2, RB)
            pi = lax.rem(i, IB)
            gi = lax.rem(i + 2, IB)
            ni = lax.rem(i + 3, IB)

            @pl.when(i + 2 < ITERS)
            def _():
                pltpu.make_async_copy(e_hbm.at[wid, i + 2], ibuf.at[gi],
                                      isem.at[gi]).wait()

                @pl.when(i >= 2)
                def _():
                    pltpu.make_async_copy(rows_v.at[g2],
                                          accum_sh.at[ibuf.at[gi, 1]],
                                          ssem.at[g2]).wait()

                pltpu.async_copy(h_hbm.at[ibuf.at[gi, 0]], rows_v.at[g2],
                                 gsem.at[g2])

            pltpu.make_async_copy(h_hbm.at[ibuf.at[pi, 0]], rows_v.at[p],
                                  gsem.at[p]).wait()
            pltpu.async_copy(rows_v.at[p], accum_sh.at[ibuf.at[pi, 1]],
                             ssem.at[p], add=True)

            @pl.when(i + 3 < ITERS)
            def _():
                pltpu.async_copy(e_hbm.at[wid, i + 3], ibuf.at[ni], isem.at[ni])

            return carry

        lax.fori_loop(0, ITERS, step, 0)

        def sdrain(j, carry):
            r = lax.rem(j, RB)
            pltpu.make_async_copy(rows_v.at[r], accum_sh.at[ibuf.at[0, 1]],
                                  ssem.at[r]).wait()
            return carry

        lax.fori_loop(0, RB, sdrain, 0)
        plsc.subcore_barrier()
        pltpu.sync_copy(accum_sh.at[pl.ds(s * RPT, RPT)], out_hbm.at[c, s])

    return agg_kernel


def _norm_from(dp0, dp1):
    deg = dp0 + dp1
    return jnp.where(deg > 0, lax.rsqrt(jnp.maximum(deg, 1.0)), 0.0)


def _tc_linear(N, D, NPAD, Bn):
    def body(x_ref, w_ref, b_ref, dp_ref, out_ref):
        norm = _norm_from(dp_ref[:, 0], dp_ref[:, 1])
        h = lax.dot_general(x_ref[...], w_ref[...], (((1,), (1,)), ((), ())),
                            preferred_element_type=jnp.float32)
        out_ref[...] = (h + b_ref[...]) * norm[:, None]

    return pl.pallas_call(
        body,
        grid=(N // Bn,),
        in_specs=[
            pl.BlockSpec((Bn, D), lambda i: (i, 0)),
            pl.BlockSpec((D, D), lambda i: (0, 0)),
            pl.BlockSpec((1, D), lambda i: (0, 0)),
            pl.BlockSpec((Bn, NC), lambda i: (i, 0)),
        ],
        out_specs=pl.BlockSpec((Bn, D), lambda i: (i, 0)),
        out_shape=jax.ShapeDtypeStruct((N, D), jnp.float32),
    )


def _tc_finish(N, D, NPAD, Bn):
    def body(ap_ref, dp_ref, out_ref):
        norm = _norm_from(dp_ref[:, 0], dp_ref[:, 1])
        a = ap_ref[0] + ap_ref[1]
        out_ref[...] = jnp.maximum(a * norm[:, None], 0.0)

    return pl.pallas_call(
        body,
        grid=(N // Bn,),
        in_specs=[
            pl.BlockSpec((NC, Bn, D), lambda i: (0, i, 0)),
            pl.BlockSpec((Bn, NC), lambda i: (i, 0)),
        ],
        out_specs=pl.BlockSpec((Bn, D), lambda i: (i, 0)),
        out_shape=jax.ShapeDtypeStruct((N, D), jnp.float32),
    )


def kernel(features, edge_index, W, b):
    N, D = features.shape
    E = edge_index.shape[1]
    NW = NC * NS
    E_PER = E // NW
    ITERS = E_PER // K
    NPAD = ((N + NS * LANES - 1) // (NS * LANES)) * (NS * LANES)
    RPT = NPAD // NS
    Bn = 1000

    edge = edge_index.astype(jnp.int32)
    dst3 = edge[1].reshape(NW, ITERS, K)
    e4 = edge.reshape(2, NW, ITERS, K).transpose(1, 2, 0, 3)

    degp = _sc_degree(NPAD, RPT, ITERS)(dst3)
    dp = degp.reshape(NC, NPAD).T
    h = _tc_linear(N, D, NPAD, Bn)(features, W, b.reshape(1, D), dp)
    accp = _sc_aggregate(NPAD, RPT, ITERS, D)(h, e4)
    ap = accp.reshape(NC, NPAD, D)
    out = _tc_finish(N, D, NPAD, Bn)(ap, dp)
    return out


# speedup vs baseline: 29.5872x; 1.0830x over previous
import functools

import jax
import jax.numpy as jnp
from jax import lax
from jax.experimental import pallas as pl
from jax.experimental.pallas import tpu as pltpu
from jax.experimental.pallas import tpu_sc as plsc

NC = 2
NS = 16
LANES = 16
K = 80
RB = 4
IB = 5


def _sc_degree(NPAD, RPT, ITERS):
    mesh = plsc.VectorSubcoreMesh(core_axis_name="c", subcore_axis_name="s")

    @functools.partial(
        pl.kernel,
        out_type=jax.ShapeDtypeStruct((NC, NS, RPT), jnp.float32),
        mesh=mesh,
        scratch_types=[
            pltpu.VMEM((ITERS, 1, K), jnp.int32),
            pltpu.VMEM((K,), jnp.float32),
            pltpu.VMEM((RPT,), jnp.float32),
            pltpu.VMEM_SHARED((NPAD,), jnp.float32),
            pltpu.SemaphoreType.DMA,
        ],
    )
    def deg_kernel(e_hbm, out_hbm, idxb, ones_v, zvec_v, deg_sh, ssem):
        c = lax.axis_index("c")
        s = lax.axis_index("s")
        wid = s * NC + c

        pltpu.sync_copy(e_hbm.at[1, wid], idxb)

        def zfill(i, carry):
            zvec_v[pl.ds(i * LANES, LANES)] = jnp.zeros((LANES,), jnp.float32)
            return carry

        lax.fori_loop(0, RPT // LANES, zfill, 0)
        pltpu.sync_copy(zvec_v, deg_sh.at[pl.ds(s * RPT, RPT)])
        for j in range(K // LANES):
            ones_v[pl.ds(j * LANES, LANES)] = jnp.ones((LANES,), jnp.float32)
        plsc.subcore_barrier()

        def fire(i, carry):
            pltpu.async_copy(ones_v, deg_sh.at[idxb.at[i, 0]], ssem, add=True)
            return carry

        lax.fori_loop(0, ITERS, fire, 0)

        def drain(i, carry):
            pltpu.make_async_copy(ones_v, deg_sh.at[idxb.at[i, 0]], ssem).wait()
            return carry

        lax.fori_loop(0, ITERS, drain, 0)
        plsc.subcore_barrier()
        pltpu.sync_copy(deg_sh.at[pl.ds(s * RPT, RPT)], out_hbm.at[c, s])

    return deg_kernel


def _sc_aggregate(NPAD, RPT, ITERS, D):
    mesh = plsc.VectorSubcoreMesh(core_axis_name="c", subcore_axis_name="s")

    @functools.partial(
        pl.kernel,
        out_type=jax.ShapeDtypeStruct((NC, NS, RPT, D), jnp.float32),
        mesh=mesh,
        scratch_types=[
            pltpu.VMEM((IB, 2, 1, K), jnp.int32),
            pltpu.VMEM((RB, K, D), jnp.float32),
            pltpu.VMEM_SHARED((NPAD, D), jnp.float32),
            pltpu.SemaphoreType.DMA((IB,)),
            pltpu.SemaphoreType.DMA((RB,)),
            pltpu.SemaphoreType.DMA((RB,)),
        ],
    )
    def agg_kernel(h_hbm, e_hbm, out_hbm, ibuf, rows_v, accum_sh, isem, gsem, ssem):
        c = lax.axis_index("c")
        s = lax.axis_index("s")
        wid = s * NC + c

        def zfill(r, carry):
            for j in range(D // LANES):
                rows_v[0, r, pl.ds(j * LANES, LANES)] = jnp.zeros((LANES,), jnp.float32)
            return carry

        lax.fori_loop(0, K, zfill, 0)

        def zcopy(k, carry):
            pltpu.sync_copy(rows_v.at[0], accum_sh.at[pl.ds(s * RPT + k * K, K)])
            return carry

        lax.fori_loop(0, RPT // K, zcopy, 0)
        plsc.subcore_barrier()

        pltpu.sync_copy(e_hbm.at[:, wid, 0], ibuf.at[0])
        pltpu.sync_copy(e_hbm.at[:, wid, 1], ibuf.at[1])
        pltpu.async_copy(h_hbm.at[ibuf.at[0, 0, 0]], rows_v.at[0], gsem.at[0])
        pltpu.async_copy(h_hbm.at[ibuf.at[1, 0, 0]], rows_v.at[1], gsem.at[1])
        pltpu.async_copy(e_hbm.at[:, wid, 2], ibuf.at[2], isem.at[2])

        def step(i, carry):
            p = lax.rem(i, RB)
            g2 = lax.rem(i + 2, RB)
            pi = lax.rem(i, IB)
            gi = lax.rem(i + 2, IB)
            ni = lax.rem(i + 3, IB)

            @pl.when(i + 2 < ITERS)
            def _():
                pltpu.make_async_copy(e_hbm.at[:, wid, i + 2], ibuf.at[gi],
                                      isem.at[gi]).wait()

                @pl.when(i >= 2)
                def _():
                    pltpu.make_async_copy(rows_v.at[g2],
                                          accum_sh.at[ibuf.at[gi, 1, 0]],
                                          ssem.at[g2]).wait()

                pltpu.async_copy(h_hbm.at[ibuf.at[gi, 0, 0]], rows_v.at[g2],
                                 gsem.at[g2])

            pltpu.make_async_copy(h_hbm.at[ibuf.at[pi, 0, 0]], rows_v.at[p],
                                  gsem.at[p]).wait()
            pltpu.async_copy(rows_v.at[p], accum_sh.at[ibuf.at[pi, 1, 0]],
                             ssem.at[p], add=True)

            @pl.when(i + 3 < ITERS)
            def _():
                pltpu.async_copy(e_hbm.at[:, wid, i + 3], ibuf.at[ni], isem.at[ni])

            return carry

        lax.fori_loop(0, ITERS, step, 0)

        def sdrain(j, carry):
            r = lax.rem(j, RB)
            pltpu.make_async_copy(rows_v.at[r], accum_sh.at[ibuf.at[0, 1, 0]],
                                  ssem.at[r]).wait()
            return carry

        lax.fori_loop(0, RB, sdrain, 0)
        plsc.subcore_barrier()
        pltpu.sync_copy(accum_sh.at[pl.ds(s * RPT, RPT)], out_hbm.at[c, s])

    return agg_kernel


def _norm_from(dp0, dp1):
    deg = dp0 + dp1
    return jnp.where(deg > 0, lax.rsqrt(jnp.maximum(deg, 1.0)), 0.0)


def _tc_linear(N, D, NPAD, Bn):
    def body(x_ref, w_ref, b_ref, dp_ref, out_ref):
        i = pl.program_id(0)
        norm = _norm_from(dp_ref[0, pl.ds(i * Bn, Bn)], dp_ref[1, pl.ds(i * Bn, Bn)])
        h = lax.dot_general(x_ref[...], w_ref[...], (((1,), (1,)), ((), ())),
                            preferred_element_type=jnp.float32)
        out_ref[...] = (h + b_ref[...]) * norm[:, None]

    return pl.pallas_call(
        body,
        grid=(NPAD // Bn,),
        in_specs=[
            pl.BlockSpec((Bn, D), lambda i: (i, 0)),
            pl.BlockSpec((D, D), lambda i: (0, 0)),
            pl.BlockSpec((1, D), lambda i: (0, 0)),
            pl.BlockSpec((NC, NPAD), lambda i: (0, 0)),
        ],
        out_specs=pl.BlockSpec((Bn, D), lambda i: (i, 0)),
        out_shape=jax.ShapeDtypeStruct((N, D), jnp.float32),
    )


def _tc_finish(N, D, NPAD, Bn):
    def body(ap_ref, dp_ref, out_ref):
        i = pl.program_id(0)
        norm = _norm_from(dp_ref[0, pl.ds(i * Bn, Bn)], dp_ref[1, pl.ds(i * Bn, Bn)])
        a = ap_ref[0] + ap_ref[1]
        out_ref[...] = jnp.maximum(a * norm[:, None], 0.0)

    return pl.pallas_call(
        body,
        grid=(NPAD // Bn,),
        in_specs=[
            pl.BlockSpec((NC, Bn, D), lambda i: (0, i, 0)),
            pl.BlockSpec((NC, NPAD), lambda i: (0, 0)),
        ],
        out_specs=pl.BlockSpec((Bn, D), lambda i: (i, 0)),
        out_shape=jax.ShapeDtypeStruct((N, D), jnp.float32),
    )


def kernel(features, edge_index, W, b):
    N, D = features.shape
    E = edge_index.shape[1]
    NW = NC * NS
    E_PER = E // NW
    ITERS = E_PER // K
    NPAD = ((N + NS * LANES - 1) // (NS * LANES)) * (NS * LANES)
    RPT = NPAD // NS
    Bn = 1024

    e5 = edge_index.astype(jnp.int32).reshape(2, NW, ITERS, 1, K)

    degp = _sc_degree(NPAD, RPT, ITERS)(e5)
    dp = degp.reshape(NC, NPAD)
    h = _tc_linear(N, D, NPAD, Bn)(features, W, b.reshape(1, D), dp)
    accp = _sc_aggregate(NPAD, RPT, ITERS, D)(h, e5)
    ap = accp.reshape(NC, NPAD, D)
    out = _tc_finish(N, D, NPAD, Bn)(ap, dp)
    return out
